# baseline (speedup 1.0000x reference)
import functools

import numpy as np

import jax
import jax.numpy as jnp
from jax import lax
from jax.experimental import pallas as pl
from jax.experimental.pallas import tpu as pltpu
from jax.experimental.pallas import tpu_sc as plsc

_CROP = 7
_SIZES = (64, 32, 16, 8, 4)
_BASES = (0, 4096, 5120, 5376, 5440)
_ROWS_PER_BATCH = 5456
_CHUNK = 32
_NUM_TILES = 32


def _point_indices_weights(image_shape, boxes):
    img = image_shape.astype(jnp.float32)
    x1 = boxes[..., 0]
    y1 = boxes[..., 1]
    x2 = boxes[..., 2]
    y2 = boxes[..., 3]
    w = x2 - x1
    h = y2 - y1
    size = jnp.sqrt(w * h)
    levels = jnp.clip(jnp.floor(1.0 + jnp.log2(size / 224.0 + 1e-7)), 0.0, 4.0)
    li = levels.astype(jnp.int32)
    si = jnp.int32(_SIZES[0]) >> li
    s = si.astype(jnp.float32)
    base = (jnp.int32(16384) - 4 * si * si) // 3

    y1n = y1 / img[1] * s / (s - 1.0)
    x1n = x1 / img[2] * s / (s - 1.0)
    y2n = (y2 / img[1] * s - 1.0) / (s - 1.0)
    x2n = (x2 / img[2] * s - 1.0) / (s - 1.0)
    hf = s - 1.0
    t = jnp.arange(_CROP, dtype=jnp.float32)[None, :, None] / jnp.float32(
        _CROP - 1)
    ys = (y1n[:, None, :] + t * (y2n - y1n)[:, None, :]) * hf[:, None, :]
    xs = (x1n[:, None, :] + t * (x2n - x1n)[:, None, :]) * hf[:, None, :]
    valid_y = (ys >= 0.0) & (ys <= hf[:, None, :])
    valid_x = (xs >= 0.0) & (xs <= hf[:, None, :])
    y0f = jnp.floor(ys)
    x0f = jnp.floor(xs)
    wy = ys - y0f
    wx = xs - x0f
    smax = si[:, None, :] - 1
    y0i = jnp.clip(y0f.astype(jnp.int32), 0, smax)
    y1i = jnp.clip(y0i + 1, 0, smax)
    x0i = jnp.clip(x0f.astype(jnp.int32), 0, smax)
    x1i = jnp.clip(x0i + 1, 0, smax)

    b_arange = jnp.arange(boxes.shape[0], dtype=jnp.int32)
    rowbase = (b_arange[:, None] * _ROWS_PER_BATCH + base)[:, None, None, :]
    s_bc = si[:, None, None, :]
    yy0 = y0i[:, :, None, :] * s_bc + rowbase
    yy1 = y1i[:, :, None, :] * s_bc + rowbase
    xx0 = x0i[:, None, :, :]
    xx1 = x1i[:, None, :, :]
    idx = jnp.stack(
        [yy0 + xx0, yy0 + xx1, yy1 + xx0, yy1 + xx1])

    m = (valid_y[:, :, None, :] & valid_x[:, None, :, :]).astype(jnp.float32)
    wy_p = wy[:, :, None, :]
    wx_q = wx[:, None, :, :]
    wgt = jnp.stack(
        [
            (1.0 - wy_p) * (1.0 - wx_q) * m,
            (1.0 - wy_p) * wx_q * m,
            wy_p * (1.0 - wx_q) * m,
            wy_p * wx_q * m,
        ])
    return idx, wgt


@functools.lru_cache(maxsize=None)
def _make_sc_call(n_points, n_real, n_chan):
    pts_per_tile = n_points // _NUM_TILES
    nchunks = pts_per_tile // _CHUNK
    assert nchunks % 2 == 0
    npt = pts_per_tile * 4
    nv = n_chan // 16
    mesh = plsc.VectorSubcoreMesh(core_axis_name="c", subcore_axis_name="s")

    @functools.partial(
        pl.kernel,
        mesh=mesh,
        out_type=jax.ShapeDtypeStruct((n_real, n_chan), jnp.float32),
        scratch_types=[
            pltpu.VMEM((4, pts_per_tile + 16), jnp.int32),
            pltpu.VMEM((4 * pts_per_tile + 16,), jnp.float32),
            pltpu.VMEM((_CHUNK * 4,), jnp.int32),
            pltpu.VMEM((_CHUNK * 4,), jnp.int32),
            pltpu.VMEM((_CHUNK * 4, n_chan // 2), jnp.int32),
            pltpu.VMEM((_CHUNK * 4, n_chan // 2), jnp.int32),
            pltpu.VMEM((_CHUNK, n_chan), jnp.float32),
            pltpu.VMEM((_CHUNK, n_chan), jnp.float32),
            pltpu.SemaphoreType.DMA,
            pltpu.SemaphoreType.DMA,
            pltpu.SemaphoreType.DMA,
            pltpu.SemaphoreType.DMA,
        ],
    )
    def roi_gather_combine(idx_hbm, wgt_hbm, table_hbm, out_hbm,
                           idx_v, wgt_v, idxb0, idxb1,
                           rows0, rows1, out0, out1,
                           gsem0, gsem1, osem0, osem1):
        wid = lax.axis_index("s") * 2 + lax.axis_index("c")
        pt0 = wid * pts_per_tile
        idxbs = (idxb0, idxb1)
        rows = (rows0, rows1)
        outs = (out0, out1)
        gsems = (gsem0, gsem1)
        osems = (osem0, osem1)

        for k in range(4):
            pltpu.sync_copy(
                idx_hbm.at[pl.ds(k * n_points + pt0, pts_per_tile)],
                idx_v.at[k, pl.ds(0, pts_per_tile)])
            pltpu.sync_copy(
                wgt_hbm.at[pl.ds(k * n_points + pt0, pts_per_tile)],
                wgt_v.at[pl.ds(k * pts_per_tile, pts_per_tile)])

        def gstart(c, buf):
            for k in range(4):
                for j in range(_CHUNK // 16):
                    idxbs[buf][pl.ds(k * _CHUNK + j * 16, 16)] = (
                        idx_v[k, pl.ds(c * _CHUNK + j * 16, 16)])
            pltpu.async_copy(table_hbm.at[idxbs[buf]], rows[buf], gsems[buf])

        def gwait(c, buf):
            pltpu.make_async_copy(table_hbm.at[idxbs[buf]], rows[buf],
                                  gsems[buf]).wait()

        def ostart(c, buf):
            base = pt0 + c * _CHUNK

            @pl.when(base + _CHUNK <= n_real)
            def _():
                pltpu.async_copy(outs[buf], out_hbm.at[pl.ds(base, _CHUNK)],
                                 osems[buf])

            @pl.when(jnp.logical_and(base < n_real, base + _CHUNK > n_real))
            def _():
                pltpu.async_copy(outs[buf].at[pl.ds(0, _CHUNK // 2)],
                                 out_hbm.at[pl.ds(base, _CHUNK // 2)],
                                 osems[buf])

        def owait(c, buf):
            base = pt0 + c * _CHUNK
            cge = c >= 0

            @pl.when(jnp.logical_and(cge, base + _CHUNK <= n_real))
            def _():
                pltpu.make_async_copy(outs[buf],
                                      out_hbm.at[pl.ds(base, _CHUNK)],
                                      osems[buf]).wait()

            @pl.when(jnp.logical_and(
                cge, jnp.logical_and(base < n_real, base + _CHUNK > n_real)))
            def _():
                pltpu.make_async_copy(outs[buf].at[pl.ds(0, _CHUNK // 2)],
                                      out_hbm.at[pl.ds(base, _CHUNK // 2)],
                                      osems[buf]).wait()

        def compute(c, buf):
            rbuf = rows[buf]
            obuf = outs[buf]

            def pt_body(i, carry):
                acc = [jnp.zeros((16,), jnp.float32)] * nv
                for k in range(4):
                    wv = wgt_v[pl.ds(k * pts_per_tile + c * _CHUNK + i, 16)]
                    wk = jnp.full((16,), wv[0], jnp.float32)
                    for j in range(nv // 2):
                        vi = rbuf[k * _CHUNK + i, pl.ds(j * 16, 16)]
                        fe = lax.bitcast_convert_type(vi << 16, jnp.float32)
                        fo = lax.bitcast_convert_type(vi, jnp.float32)
                        acc[2 * j] = acc[2 * j] + fe * wk
                        acc[2 * j + 1] = acc[2 * j + 1] + fo * wk
                for j in range(nv):
                    obuf[i, pl.ds(j * 16, 16)] = acc[j]
                return carry

            lax.fori_loop(0, _CHUNK, pt_body, 0, unroll=False)

        gstart(0, 0)

        def body2(h, carry):
            c0 = 2 * h
            c1 = c0 + 1
            gstart(c1, 1)
            gwait(c0, 0)
            owait(c0 - 2, 0)
            compute(c0, 0)
            ostart(c0, 0)

            @pl.when(c0 + 2 < nchunks)
            def _():
                gstart(c0 + 2, 0)

            gwait(c1, 1)
            owait(c1 - 2, 1)
            compute(c1, 1)
            ostart(c1, 1)
            return carry

        lax.fori_loop(0, nchunks // 2, body2, 0, unroll=False)
        owait(nchunks - 2, 0)
        owait(nchunks - 1, 1)

    return roi_gather_combine


def kernel(image_shape, boxes, scores, fpn0, fpn1, fpn2, fpn3, fpn4):
    del scores
    b, n = boxes.shape[:2]
    c = fpn0.shape[-1]
    fpns = (fpn0, fpn1, fpn2, fpn3, fpn4)
    table = jnp.concatenate([f.reshape(b, -1, c) for f in fpns], axis=1)
    table = table.reshape(b * _ROWS_PER_BATCH, c)
    perm = np.arange(c).reshape(-1, 2, 16).transpose(0, 2, 1).reshape(-1)
    table = table[:, perm].astype(jnp.bfloat16)
    table = jax.lax.bitcast_convert_type(
        table.reshape(b * _ROWS_PER_BATCH, c // 2, 2), jnp.int32)

    idx, wgt = _point_indices_weights(image_shape, boxes)
    n_pts = _CROP * _CROP
    n_real = b * n * n_pts
    grain = _NUM_TILES * _CHUNK * 2
    n_points = ((n_real + grain - 1) // grain) * grain
    idx = jnp.pad(idx.reshape(4, n_real), ((0, 0), (0, n_points - n_real)))
    wgt = jnp.pad(wgt.reshape(4, n_real), ((0, 0), (0, n_points - n_real)))

    call = _make_sc_call(n_points, n_real, c)
    out = call(idx.reshape(4 * n_points), wgt.reshape(4 * n_points), table)
    return out.reshape(b, _CROP, _CROP, n, c).transpose(0, 3, 1, 2, 4)


# speedup vs baseline: 1.3182x; 1.3182x over previous
import functools

import numpy as np

import jax
import jax.numpy as jnp
from jax import lax
from jax.experimental import pallas as pl
from jax.experimental.pallas import tpu as pltpu
from jax.experimental.pallas import tpu_sc as plsc

_CROP = 7
_SIZES = (64, 32, 16, 8, 4)
_BASES = (0, 4096, 5120, 5376, 5440)
_ROWS_PER_BATCH = 5456
_CHUNK = 16
_NBUF = 4
_NUM_TILES = 32


def _point_indices_weights(image_shape, boxes):
    img = image_shape.astype(jnp.float32)
    x1 = boxes[..., 0]
    y1 = boxes[..., 1]
    x2 = boxes[..., 2]
    y2 = boxes[..., 3]
    w = x2 - x1
    h = y2 - y1
    size = jnp.sqrt(w * h)
    levels = jnp.clip(jnp.floor(1.0 + jnp.log2(size / 224.0 + 1e-7)), 0.0, 4.0)
    li = levels.astype(jnp.int32)
    si = jnp.int32(_SIZES[0]) >> li
    s = si.astype(jnp.float32)
    base = (jnp.int32(16384) - 4 * si * si) // 3

    y1n = y1 / img[1] * s / (s - 1.0)
    x1n = x1 / img[2] * s / (s - 1.0)
    y2n = (y2 / img[1] * s - 1.0) / (s - 1.0)
    x2n = (x2 / img[2] * s - 1.0) / (s - 1.0)
    hf = s - 1.0
    t = jnp.arange(_CROP, dtype=jnp.float32)[None, :, None] / jnp.float32(
        _CROP - 1)
    ys = (y1n[:, None, :] + t * (y2n - y1n)[:, None, :]) * hf[:, None, :]
    xs = (x1n[:, None, :] + t * (x2n - x1n)[:, None, :]) * hf[:, None, :]
    valid_y = (ys >= 0.0) & (ys <= hf[:, None, :])
    valid_x = (xs >= 0.0) & (xs <= hf[:, None, :])
    y0f = jnp.floor(ys)
    x0f = jnp.floor(xs)
    wy = ys - y0f
    wx = xs - x0f
    smax = si[:, None, :] - 1
    y0i = jnp.clip(y0f.astype(jnp.int32), 0, smax)
    y1i = jnp.clip(y0i + 1, 0, smax)
    x0i = jnp.clip(x0f.astype(jnp.int32), 0, smax)
    x1i = jnp.clip(x0i + 1, 0, smax)

    b_arange = jnp.arange(boxes.shape[0], dtype=jnp.int32)
    rowbase = (b_arange[:, None] * _ROWS_PER_BATCH + base)[:, None, None, :]
    s_bc = si[:, None, None, :]
    yy0 = y0i[:, :, None, :] * s_bc + rowbase
    yy1 = y1i[:, :, None, :] * s_bc + rowbase
    xx0 = x0i[:, None, :, :]
    xx1 = x1i[:, None, :, :]
    idx = jnp.stack(
        [yy0 + xx0, yy0 + xx1, yy1 + xx0, yy1 + xx1])

    m = (valid_y[:, :, None, :] & valid_x[:, None, :, :]).astype(jnp.float32)
    wy_p = wy[:, :, None, :]
    wx_q = wx[:, None, :, :]
    wgt = jnp.stack(
        [
            (1.0 - wy_p) * (1.0 - wx_q) * m,
            (1.0 - wy_p) * wx_q * m,
            wy_p * (1.0 - wx_q) * m,
            wy_p * wx_q * m,
        ])
    return idx, wgt


@functools.lru_cache(maxsize=None)
def _make_sc_call(n_points, n_real, n_chan):
    pts_per_tile = n_points // _NUM_TILES
    nchunks = pts_per_tile // _CHUNK
    assert nchunks % _NBUF == 0
    assert n_real % _CHUNK == 0
    npt = pts_per_tile * 4
    nv = n_chan // 16
    mesh = plsc.VectorSubcoreMesh(core_axis_name="c", subcore_axis_name="s")

    @functools.partial(
        pl.kernel,
        mesh=mesh,
        out_type=jax.ShapeDtypeStruct((n_real, n_chan), jnp.float32),
        scratch_types=(
            [pltpu.VMEM((4, pts_per_tile + 16), jnp.int32)]
            + [pltpu.VMEM((4 * pts_per_tile + 16,), jnp.float32)]
            + [pltpu.VMEM((_CHUNK * 4,), jnp.int32)] * _NBUF
            + [pltpu.VMEM((_CHUNK * 4, n_chan), jnp.float32)] * _NBUF
            + [pltpu.VMEM((_CHUNK, n_chan), jnp.float32)] * _NBUF
            + [pltpu.SemaphoreType.DMA] * (2 * _NBUF)
        ),
    )
    def roi_gather_combine(idx_hbm, wgt_hbm, table_hbm, out_hbm,
                           idx_v, wgt_v, *bufs):
        wid = lax.axis_index("s") * 2 + lax.axis_index("c")
        pt0 = wid * pts_per_tile
        idxbs = bufs[0:_NBUF]
        rows = bufs[_NBUF:2 * _NBUF]
        outs = bufs[2 * _NBUF:3 * _NBUF]
        gsems = bufs[3 * _NBUF:4 * _NBUF]
        osems = bufs[4 * _NBUF:5 * _NBUF]

        for k in range(4):
            pltpu.sync_copy(
                idx_hbm.at[pl.ds(k * n_points + pt0, pts_per_tile)],
                idx_v.at[k, pl.ds(0, pts_per_tile)])
            pltpu.sync_copy(
                wgt_hbm.at[pl.ds(k * n_points + pt0, pts_per_tile)],
                wgt_v.at[pl.ds(k * pts_per_tile, pts_per_tile)])

        def gstart(c, buf):
            for k in range(4):
                for j in range(_CHUNK // 16):
                    idxbs[buf][pl.ds(k * _CHUNK + j * 16, 16)] = (
                        idx_v[k, pl.ds(c * _CHUNK + j * 16, 16)])
            pltpu.async_copy(table_hbm.at[idxbs[buf]], rows[buf], gsems[buf])

        def gwait(c, buf):
            pltpu.make_async_copy(table_hbm.at[idxbs[buf]], rows[buf],
                                  gsems[buf]).wait()

        def ostart(c, buf):
            base = pt0 + c * _CHUNK

            @pl.when(base < n_real)
            def _():
                pltpu.async_copy(outs[buf], out_hbm.at[pl.ds(base, _CHUNK)],
                                 osems[buf])

        def owait(c, buf):
            base = pt0 + c * _CHUNK

            @pl.when(jnp.logical_and(c >= 0, base < n_real))
            def _():
                pltpu.make_async_copy(outs[buf],
                                      out_hbm.at[pl.ds(base, _CHUNK)],
                                      osems[buf]).wait()

        def compute(c, buf):
            rbuf = rows[buf]
            obuf = outs[buf]

            def pt_body(i, carry):
                acc = [jnp.zeros((16,), jnp.float32)] * nv
                for k in range(4):
                    wv = wgt_v[pl.ds(k * pts_per_tile + c * _CHUNK + i, 16)]
                    wk = jnp.full((16,), wv[0], jnp.float32)
                    for j in range(nv):
                        acc[j] = acc[j] + rbuf[k * _CHUNK + i,
                                               pl.ds(j * 16, 16)] * wk
                for j in range(nv):
                    obuf[i, pl.ds(j * 16, 16)] = acc[j]
                return carry

            lax.fori_loop(0, _CHUNK, pt_body, 0, unroll=False)

        for u in range(_NBUF - 1):
            gstart(u, u)

        def body(h, carry):
            for u in range(_NBUF):
                c = _NBUF * h + u

                @pl.when(c + _NBUF - 1 < nchunks)
                def _():
                    gstart(c + _NBUF - 1, (u + _NBUF - 1) % _NBUF)

                gwait(c, u)
                owait(c - _NBUF, u)
                compute(c, u)
                ostart(c, u)
            return carry

        lax.fori_loop(0, nchunks // _NBUF, body, 0, unroll=False)
        for u in range(_NBUF):
            owait(nchunks - _NBUF + u, u)

    return roi_gather_combine


def kernel(image_shape, boxes, scores, fpn0, fpn1, fpn2, fpn3, fpn4):
    del scores
    b, n = boxes.shape[:2]
    c = fpn0.shape[-1]
    fpns = (fpn0, fpn1, fpn2, fpn3, fpn4)
    table = jnp.concatenate([f.reshape(b, -1, c) for f in fpns], axis=1)
    table = table.reshape(b * _ROWS_PER_BATCH, c)

    idx, wgt = _point_indices_weights(image_shape, boxes)
    n_pts = _CROP * _CROP
    n_real = b * n * n_pts
    grain = _NUM_TILES * _CHUNK * _NBUF
    n_points = ((n_real + grain - 1) // grain) * grain
    idx = jnp.pad(idx.reshape(4, n_real), ((0, 0), (0, n_points - n_real)))
    wgt = jnp.pad(wgt.reshape(4, n_real), ((0, 0), (0, n_points - n_real)))

    call = _make_sc_call(n_points, n_real, c)
    out = call(idx.reshape(4 * n_points), wgt.reshape(4 * n_points), table)
    return out.reshape(b, _CROP, _CROP, n, c).transpose(0, 3, 1, 2, 4)
